# baseline (speedup 1.0000x reference)
import functools

import jax
import jax.numpy as jnp
from jax import lax
from jax.experimental import pallas as pl
from jax.experimental.pallas import tpu as pltpu
from jax.experimental.pallas import tpu_sc as plsc

N, D, G, GD, BIG, BIGGER, OUT, K = 4096, 128, 16, 16, 256, 512, 1, 16
EPS = 1e-5

RB = 256
NB = 256
SC_WORKERS = 32
SC_CHUNK = 256


def _prep_body(x_ref, w1_ref, g_ref, b_ref, xn_ref, a_ref):
    x = x_ref[...]
    mu = jnp.mean(x, axis=0, keepdims=True)
    xc = x - mu
    var = jnp.mean(xc * xc, axis=0, keepdims=True)
    xn = xc / jnp.sqrt(var + EPS) * g_ref[...] + b_ref[...]
    xn_ref[...] = xn
    w1t = w1_ref[:D, :]
    w1b = w1_ref[D:, :]
    a_ref[...] = jnp.dot(xn, w1t - w1b, preferred_element_type=jnp.float32)


def _prep(x, W1, bn_gamma, bn_beta):
    return pl.pallas_call(
        _prep_body,
        out_shape=(
            jax.ShapeDtypeStruct((N, D), jnp.float32),
            jax.ShapeDtypeStruct((N, BIG), jnp.float32),
        ),
    )(x, W1, bn_gamma.reshape(1, D), bn_beta.reshape(1, D))


WIN = 1280
WB = 256


def _topk_win_body(half, c0_ref, xnb_ref, x0, x1, x2, x3, x4, bfb_ref,
                   r0, r1, r2, r3, r4, idx_ref):
    i = half * HB + pl.program_id(0)
    c0 = c0_ref[i] * WB
    xnw = jnp.concatenate([x0[...], x1[...], x2[...], x3[...], x4[...]], axis=0)
    bfr = jnp.concatenate([r0[...], r1[...], r2[...], r3[...], r4[...]], axis=1)
    xnb = xnb_ref[...]
    sqb = jnp.sum(xnb * xnb, axis=1, keepdims=True)
    ysq = xnw * xnw
    sqr = jnp.dot(jnp.ones((1, D), jnp.float32), ysq.T,
                  preferred_element_type=jnp.float32)
    dist = sqb + sqr - 2.0 * jnp.dot(xnb, xnw.T, preferred_element_type=jnp.float32)
    cross = bfb_ref[...] != bfr
    d = jnp.where(cross, 1e10, dist)
    iota = lax.broadcasted_iota(jnp.int32, (RB, WIN), 1)
    inf = jnp.float32(jnp.inf)
    for k in range(K):
        m = jnp.min(d, axis=1, keepdims=True)
        amin = jnp.min(jnp.where(d == m, iota, WIN), axis=1, keepdims=True)
        idx_ref[:, k:k + 1] = amin + c0
        d = jnp.where(iota == amin, inf, d)


HB = (N // RB) // 2


def _topk_win(xn, batchf_col, batchf_row, c0_blocks, half):
    grid_spec = pltpu.PrefetchScalarGridSpec(
        num_scalar_prefetch=1,
        grid=(HB,),
        in_specs=[
            pl.BlockSpec((RB, D), lambda i, c0: (half * HB + i, 0)),
        ] + [
            pl.BlockSpec((WB, D), functools.partial(
                lambda j, i, c0: (c0[half * HB + i] + j, 0), j))
            for j in range(5)
        ] + [
            pl.BlockSpec((RB, 1), lambda i, c0: (half * HB + i, 0)),
        ] + [
            pl.BlockSpec((1, WB), functools.partial(
                lambda j, i, c0: (0, c0[half * HB + i] + j), j))
            for j in range(5)
        ],
        out_specs=pl.BlockSpec((RB, K), lambda i, c0: (i, 0)),
    )
    return pl.pallas_call(
        functools.partial(_topk_win_body, half),
        grid_spec=grid_spec,
        out_shape=jax.ShapeDtypeStruct((N // 2, K), jnp.int32),
    )(c0_blocks, xn, xn, xn, xn, xn, xn, batchf_col,
      batchf_row, batchf_row, batchf_row, batchf_row, batchf_row)


def _topk_body(xnb_ref, xn_ref, bfb_ref, bfr_ref, idx_ref):
    xnb = xnb_ref[...]
    xn = xn_ref[...]
    sqb = jnp.sum(xnb * xnb, axis=1, keepdims=True)
    ysq = xn * xn
    sqr = jnp.dot(jnp.ones((1, D), jnp.float32), ysq.T,
                  preferred_element_type=jnp.float32)
    dist = sqb + sqr - 2.0 * jnp.dot(xnb, xn.T, preferred_element_type=jnp.float32)
    cross = bfb_ref[...] != bfr_ref[...]
    d = jnp.where(cross, 1e10, dist)
    iota = lax.broadcasted_iota(jnp.int32, (RB, N), 1)
    inf = jnp.float32(jnp.inf)
    for k in range(K):
        m = jnp.min(d, axis=1, keepdims=True)
        amin = jnp.min(jnp.where(d == m, iota, N), axis=1, keepdims=True)
        idx_ref[:, k:k + 1] = amin
        d = jnp.where(iota == amin, inf, d)


def _topk(xn, batchf_col, batchf_row, half):
    return pl.pallas_call(
        _topk_body,
        grid=(HB,),
        in_specs=[
            pl.BlockSpec((RB, D), lambda i: (half * HB + i, 0)),
            pl.BlockSpec((N, D), lambda i: (0, 0)),
            pl.BlockSpec((RB, 1), lambda i: (half * HB + i, 0)),
            pl.BlockSpec((1, N), lambda i: (0, 0)),
        ],
        out_specs=pl.BlockSpec((RB, K), lambda i: (i, 0)),
        out_shape=jax.ShapeDtypeStruct((N // 2, K), jnp.int32),
    )(xn, xn, batchf_col, batchf_row)


def _sc_gather_body(nrows, table_hbm, idx_hbm, out_hbm, idx_v, rows_v, sem):
    wid = lax.axis_index("s") * 2 + lax.axis_index("c")
    per_w = nrows // SC_WORKERS
    base = wid * per_w

    def chunk(c, _):
        off = pl.multiple_of(base + c * SC_CHUNK, SC_CHUNK)
        pltpu.sync_copy(idx_hbm.at[pl.ds(off, SC_CHUNK)], idx_v)
        pltpu.async_copy(table_hbm.at[idx_v], rows_v, sem).wait()
        pltpu.sync_copy(rows_v, out_hbm.at[pl.ds(off, SC_CHUNK)])
        return _

    lax.fori_loop(0, per_w // SC_CHUNK, chunk, None)


def _sc_gather(table, idx_flat):
    nrows = idx_flat.shape[0]
    mesh = plsc.VectorSubcoreMesh(core_axis_name="c", subcore_axis_name="s")
    kfn = functools.partial(
        pl.kernel,
        mesh=mesh,
        out_type=jax.ShapeDtypeStruct((nrows, D), jnp.float32),
        scratch_types=[
            pltpu.VMEM((SC_CHUNK,), jnp.int32),
            pltpu.VMEM((SC_CHUNK, D), jnp.float32),
            pltpu.SemaphoreType.DMA,
        ],
    )(functools.partial(_sc_gather_body, nrows))
    return kfn(table, idx_flat)


def _edge_body(a_ref, xj_ref, bfb_ref, w1b_ref, w2_ref, w3_ref,
               b1_ref, b2_ref, b3_ref, seg_ref, cnt_ref):
    @pl.when(pl.program_id(0) == 0)
    def _init():
        seg_ref[...] = jnp.zeros_like(seg_ref)
        cnt_ref[...] = jnp.zeros_like(cnt_ref)

    a = a_ref[...]
    a_rep = jnp.broadcast_to(a[:, None, :], (NB, K, BIG)).reshape(NB * K, BIG)
    bj = jnp.dot(xj_ref[...], w1b_ref[...], preferred_element_type=jnp.float32)
    h1 = jnp.maximum(a_rep + bj + b1_ref[...], 0.0)
    h2 = jnp.maximum(jnp.dot(h1, w2_ref[...], preferred_element_type=jnp.float32)
                     + b2_ref[...], 0.0)
    h3 = jnp.maximum(jnp.dot(h2, w3_ref[...], preferred_element_type=jnp.float32)
                     + b3_ref[...], 0.0)
    hx = jnp.mean(h3.reshape(NB, K, BIG), axis=1)
    gids = lax.broadcasted_iota(jnp.int32, (1, G), 1).astype(jnp.float32)
    onehot = (bfb_ref[...] == gids).astype(jnp.float32)
    seg_ref[...] += lax.dot_general(onehot, hx, (((0,), (0,)), ((), ())),
                                    preferred_element_type=jnp.float32)
    cnt_ref[...] += jnp.broadcast_to(
        jnp.sum(onehot, axis=0)[:, None], (G, BIG))


def _edge(A, XJ, batchf_col, W1b, W2, W3, b1, b2, b3, half):
    nb_half = (N // NB) // 2
    return pl.pallas_call(
        _edge_body,
        grid=(nb_half,),
        in_specs=[
            pl.BlockSpec((NB, BIG), lambda i: (half * nb_half + i, 0)),
            pl.BlockSpec((NB * K, D), lambda i: (i, 0)),
            pl.BlockSpec((NB, 1), lambda i: (half * nb_half + i, 0)),
            pl.BlockSpec((D, BIG), lambda i: (0, 0)),
            pl.BlockSpec((BIG, BIG), lambda i: (0, 0)),
            pl.BlockSpec((BIG, BIG), lambda i: (0, 0)),
            pl.BlockSpec((1, BIG), lambda i: (0, 0)),
            pl.BlockSpec((1, BIG), lambda i: (0, 0)),
            pl.BlockSpec((1, BIG), lambda i: (0, 0)),
        ],
        out_specs=(
            pl.BlockSpec((G, BIG), lambda i: (0, 0)),
            pl.BlockSpec((G, BIG), lambda i: (0, 0)),
        ),
        out_shape=(
            jax.ShapeDtypeStruct((G, BIG), jnp.float32),
            jax.ShapeDtypeStruct((G, BIG), jnp.float32),
        ),
    )(A, XJ, batchf_col, W1b, W2, W3,
      b1.reshape(1, BIG), b2.reshape(1, BIG), b3.reshape(1, BIG))


def _head_body(u_ref, gg_ref, gb_ref, sega_ref, segb_ref, cnta_ref, cntb_ref,
               o1w_ref, o1b_ref, o2w_ref, o2b_ref, o3w_ref, o3b_ref, out_ref):
    u = u_ref[...]
    mu = jnp.mean(u, axis=0, keepdims=True)
    uc = u - mu
    var = jnp.mean(uc * uc, axis=0, keepdims=True)
    u1 = uc / jnp.sqrt(var + EPS) * gg_ref[...] + gb_ref[...]
    u2 = (sega_ref[...] + segb_ref[...]) / jnp.maximum(
        cnta_ref[...] + cntb_ref[...], 1.0)
    uu = jnp.concatenate([u1, u2], axis=1)
    o = jnp.maximum(jnp.dot(uu, o1w_ref[...], preferred_element_type=jnp.float32)
                    + o1b_ref[...], 0.0)
    o = jnp.maximum(jnp.dot(o, o2w_ref[...], preferred_element_type=jnp.float32)
                    + o2b_ref[...], 0.0)
    out_ref[...] = jnp.dot(o, o3w_ref[...], preferred_element_type=jnp.float32) \
        + o3b_ref[...]


def _head(u, bng_gamma, bng_beta, sega, segb, cnta, cntb,
          O1, o1, O2, o2, O3, o3):
    return pl.pallas_call(
        _head_body,
        out_shape=jax.ShapeDtypeStruct((G, OUT), jnp.float32),
    )(u, bng_gamma.reshape(1, GD), bng_beta.reshape(1, GD),
      sega, segb, cnta, cntb,
      O1, o1.reshape(1, BIGGER), O2, o2.reshape(1, BIGGER), O3,
      o3.reshape(1, OUT))


def kernel(x, u, batch, bn_gamma, bn_beta, bng_gamma, bng_beta,
           W1, b1, W2, b2, W3, b3, O1, o1, O2, o2, O3, o3):
    batchf = batch.astype(jnp.float32)
    xn, A = _prep(x, W1, bn_gamma, bn_beta)

    gid = jnp.arange(G, dtype=jnp.int32)
    counts = jnp.sum(batch[None, :] == gid[:, None], axis=1)
    ends = jnp.cumsum(counts)
    starts = ends - counts
    rb_first = batch[::RB]
    rb_last = batch[RB - 1::RB]
    span = ends[rb_last] - starts[rb_first]
    win_ok = jnp.all(span <= WIN - WB) & jnp.all(counts >= K)
    c0_blocks = jnp.clip(starts[rb_first] // WB, 0, (N - WIN) // WB)

    bcol = batchf.reshape(N, 1)
    brow = batchf.reshape(1, N)
    c0b = c0_blocks.astype(jnp.int32)
    idx_a, idx_b = lax.cond(
        win_ok,
        lambda: (_topk_win(xn, bcol, brow, c0b, 0),
                 _topk_win(xn, bcol, brow, c0b, 1)),
        lambda: (_topk(xn, bcol, brow, 0),
                 _topk(xn, bcol, brow, 1)),
    )
    XJ_a = _sc_gather(xn, idx_a.reshape((N // 2) * K))
    XJ_b = _sc_gather(xn, idx_b.reshape((N // 2) * K))
    W1b = W1[D:, :]
    sega, cnta = _edge(A, XJ_a, bcol, W1b, W2, W3, b1, b2, b3, 0)
    segb, cntb = _edge(A, XJ_b, bcol, W1b, W2, W3, b1, b2, b3, 1)
    return _head(u, bng_gamma, bng_beta, sega, segb, cnta, cntb,
                 O1, o1, O2, o2, O3, o3)


# speedup vs baseline: 1.0042x; 1.0042x over previous
import functools

import jax
import jax.numpy as jnp
from jax import lax
from jax.experimental import pallas as pl
from jax.experimental.pallas import tpu as pltpu
from jax.experimental.pallas import tpu_sc as plsc

N, D, G, GD, BIG, BIGGER, OUT, K = 4096, 128, 16, 16, 256, 512, 1, 16
EPS = 1e-5

RB = 256
NB = 256
SC_WORKERS = 32
SC_CHUNK = 256


def _prep_body(x_ref, w1_ref, g_ref, b_ref, xn_ref, a_ref):
    x = x_ref[...]
    mu = jnp.mean(x, axis=0, keepdims=True)
    xc = x - mu
    var = jnp.mean(xc * xc, axis=0, keepdims=True)
    xn = xc / jnp.sqrt(var + EPS) * g_ref[...] + b_ref[...]
    xn_ref[...] = xn
    w1t = w1_ref[:D, :]
    w1b = w1_ref[D:, :]
    a_ref[...] = jnp.dot(xn, w1t - w1b, preferred_element_type=jnp.float32)


def _prep(x, W1, bn_gamma, bn_beta):
    return pl.pallas_call(
        _prep_body,
        out_shape=(
            jax.ShapeDtypeStruct((N, D), jnp.float32),
            jax.ShapeDtypeStruct((N, BIG), jnp.float32),
        ),
    )(x, W1, bn_gamma.reshape(1, D), bn_beta.reshape(1, D))


WIN = 1280
WB = 256


def _topk_win_body(half, c0_ref, xnb_ref, x0, x1, x2, x3, x4, bfb_ref,
                   r0, r1, r2, r3, r4, idx_ref):
    i = half * HB + pl.program_id(0)
    c0 = c0_ref[i] * WB
    xnw = jnp.concatenate([x0[...], x1[...], x2[...], x3[...], x4[...]], axis=0)
    bfr = jnp.concatenate([r0[...], r1[...], r2[...], r3[...], r4[...]], axis=1)
    xnb = xnb_ref[...]
    sqb = jnp.sum(xnb * xnb, axis=1, keepdims=True)
    ysq = xnw * xnw
    sqr = jnp.dot(jnp.ones((1, D), jnp.float32), ysq.T,
                  preferred_element_type=jnp.float32)
    dist = sqb + sqr - 2.0 * jnp.dot(xnb, xnw.T, preferred_element_type=jnp.float32)
    cross = bfb_ref[...] != bfr
    d = jnp.where(cross, 1e10, dist)
    iota = lax.broadcasted_iota(jnp.int32, (RB, WIN), 1)
    inf = jnp.float32(jnp.inf)
    for k in range(K):
        m = jnp.min(d, axis=1, keepdims=True)
        amin = jnp.min(jnp.where(d == m, iota, WIN), axis=1, keepdims=True)
        idx_ref[:, k:k + 1] = amin + c0
        d = jnp.where(iota == amin, inf, d)


HB = (N // RB) // 2


def _topk_win(xn, batchf_col, batchf_row, c0_blocks, half):
    grid_spec = pltpu.PrefetchScalarGridSpec(
        num_scalar_prefetch=1,
        grid=(HB,),
        in_specs=[
            pl.BlockSpec((RB, D), lambda i, c0: (half * HB + i, 0)),
        ] + [
            pl.BlockSpec((WB, D), functools.partial(
                lambda j, i, c0: (c0[half * HB + i] + j, 0), j))
            for j in range(5)
        ] + [
            pl.BlockSpec((RB, 1), lambda i, c0: (half * HB + i, 0)),
        ] + [
            pl.BlockSpec((1, WB), functools.partial(
                lambda j, i, c0: (0, c0[half * HB + i] + j), j))
            for j in range(5)
        ],
        out_specs=pl.BlockSpec((RB, K), lambda i, c0: (i, 0)),
    )
    return pl.pallas_call(
        functools.partial(_topk_win_body, half),
        grid_spec=grid_spec,
        out_shape=jax.ShapeDtypeStruct((N // 2, K), jnp.int32),
    )(c0_blocks, xn, xn, xn, xn, xn, xn, batchf_col,
      batchf_row, batchf_row, batchf_row, batchf_row, batchf_row)


def _topk_body(xnb_ref, xn_ref, bfb_ref, bfr_ref, idx_ref):
    xnb = xnb_ref[...]
    xn = xn_ref[...]
    sqb = jnp.sum(xnb * xnb, axis=1, keepdims=True)
    ysq = xn * xn
    sqr = jnp.dot(jnp.ones((1, D), jnp.float32), ysq.T,
                  preferred_element_type=jnp.float32)
    dist = sqb + sqr - 2.0 * jnp.dot(xnb, xn.T, preferred_element_type=jnp.float32)
    cross = bfb_ref[...] != bfr_ref[...]
    d = jnp.where(cross, 1e10, dist)
    iota = lax.broadcasted_iota(jnp.int32, (RB, N), 1)
    inf = jnp.float32(jnp.inf)
    for k in range(K):
        m = jnp.min(d, axis=1, keepdims=True)
        amin = jnp.min(jnp.where(d == m, iota, N), axis=1, keepdims=True)
        idx_ref[:, k:k + 1] = amin
        d = jnp.where(iota == amin, inf, d)


def _topk(xn, batchf_col, batchf_row, half):
    return pl.pallas_call(
        _topk_body,
        grid=(HB,),
        in_specs=[
            pl.BlockSpec((RB, D), lambda i: (half * HB + i, 0)),
            pl.BlockSpec((N, D), lambda i: (0, 0)),
            pl.BlockSpec((RB, 1), lambda i: (half * HB + i, 0)),
            pl.BlockSpec((1, N), lambda i: (0, 0)),
        ],
        out_specs=pl.BlockSpec((RB, K), lambda i: (i, 0)),
        out_shape=jax.ShapeDtypeStruct((N // 2, K), jnp.int32),
    )(xn, xn, batchf_col, batchf_row)


def _sc_gather_body(nrows, table_hbm, idx_hbm, out_hbm, idx_v, rows_v, sem):
    wid = lax.axis_index("s") * 2 + lax.axis_index("c")
    per_w = nrows // SC_WORKERS
    base = wid * per_w

    def chunk(c, _):
        off = pl.multiple_of(base + c * SC_CHUNK, SC_CHUNK)
        pltpu.sync_copy(idx_hbm.at[pl.ds(off, SC_CHUNK)], idx_v)
        pltpu.async_copy(table_hbm.at[idx_v], rows_v, sem).wait()
        pltpu.sync_copy(rows_v, out_hbm.at[pl.ds(off, SC_CHUNK)])
        return _

    lax.fori_loop(0, per_w // SC_CHUNK, chunk, None)


def _sc_gather(table, idx_flat):
    nrows = idx_flat.shape[0]
    mesh = plsc.VectorSubcoreMesh(core_axis_name="c", subcore_axis_name="s")
    kfn = functools.partial(
        pl.kernel,
        mesh=mesh,
        out_type=jax.ShapeDtypeStruct((nrows, D), jnp.float32),
        scratch_types=[
            pltpu.VMEM((SC_CHUNK,), jnp.int32),
            pltpu.VMEM((SC_CHUNK, D), jnp.float32),
            pltpu.SemaphoreType.DMA,
        ],
    )(functools.partial(_sc_gather_body, nrows))
    return kfn(table, idx_flat)


def _edge_body(a_ref, xj_ref, bfb_ref, w1b_ref, w2_ref, w3_ref,
               b1_ref, b2_ref, b3_ref, seg_ref, cnt_ref):
    @pl.when(pl.program_id(0) == 0)
    def _init():
        seg_ref[...] = jnp.zeros_like(seg_ref)
        cnt_ref[...] = jnp.zeros_like(cnt_ref)

    a = a_ref[...]
    a_rep = jnp.broadcast_to(a[:, None, :], (NB, K, BIG)).reshape(NB * K, BIG)
    bj = jnp.dot(xj_ref[...], w1b_ref[...], preferred_element_type=jnp.float32)
    h1 = jnp.maximum(a_rep + bj + b1_ref[...], 0.0)
    h2 = jnp.maximum(jnp.dot(h1, w2_ref[...], preferred_element_type=jnp.float32)
                     + b2_ref[...], 0.0)
    h3 = jnp.maximum(jnp.dot(h2, w3_ref[...], preferred_element_type=jnp.float32)
                     + b3_ref[...], 0.0)
    hx = jnp.mean(h3.reshape(NB, K, BIG), axis=1)
    gids = lax.broadcasted_iota(jnp.int32, (1, G), 1).astype(jnp.float32)
    onehot = (bfb_ref[...] == gids).astype(jnp.float32)
    seg_ref[...] += lax.dot_general(onehot, hx, (((0,), (0,)), ((), ())),
                                    preferred_element_type=jnp.float32)
    cnt_ref[...] += jnp.broadcast_to(
        jnp.sum(onehot, axis=0)[:, None], (G, BIG))


def _edge(A, XJ, batchf_col, W1b, W2, W3, b1, b2, b3, half):
    nb_half = (N // NB) // 2
    return pl.pallas_call(
        _edge_body,
        grid=(nb_half,),
        in_specs=[
            pl.BlockSpec((NB, BIG), lambda i: (half * nb_half + i, 0)),
            pl.BlockSpec((NB * K, D), lambda i: (i, 0)),
            pl.BlockSpec((NB, 1), lambda i: (half * nb_half + i, 0)),
            pl.BlockSpec((D, BIG), lambda i: (0, 0)),
            pl.BlockSpec((BIG, BIG), lambda i: (0, 0)),
            pl.BlockSpec((BIG, BIG), lambda i: (0, 0)),
            pl.BlockSpec((1, BIG), lambda i: (0, 0)),
            pl.BlockSpec((1, BIG), lambda i: (0, 0)),
            pl.BlockSpec((1, BIG), lambda i: (0, 0)),
        ],
        out_specs=(
            pl.BlockSpec((G, BIG), lambda i: (0, 0)),
            pl.BlockSpec((G, BIG), lambda i: (0, 0)),
        ),
        out_shape=(
            jax.ShapeDtypeStruct((G, BIG), jnp.float32),
            jax.ShapeDtypeStruct((G, BIG), jnp.float32),
        ),
    )(A, XJ, batchf_col, W1b, W2, W3,
      b1.reshape(1, BIG), b2.reshape(1, BIG), b3.reshape(1, BIG))


def _head_body(u_ref, gg_ref, gb_ref, sega_ref, segb_ref, cnta_ref, cntb_ref,
               o1w_ref, o1b_ref, o2w_ref, o2b_ref, o3w_ref, o3b_ref, out_ref):
    u = u_ref[...]
    mu = jnp.mean(u, axis=0, keepdims=True)
    uc = u - mu
    var = jnp.mean(uc * uc, axis=0, keepdims=True)
    u1 = uc / jnp.sqrt(var + EPS) * gg_ref[...] + gb_ref[...]
    u2 = (sega_ref[...] + segb_ref[...]) / jnp.maximum(
        cnta_ref[...] + cntb_ref[...], 1.0)
    uu = jnp.concatenate([u1, u2], axis=1)
    o = jnp.maximum(jnp.dot(uu, o1w_ref[...], preferred_element_type=jnp.float32)
                    + o1b_ref[...], 0.0)
    o = jnp.maximum(jnp.dot(o, o2w_ref[...], preferred_element_type=jnp.float32)
                    + o2b_ref[...], 0.0)
    out_ref[...] = jnp.dot(o, o3w_ref[...], preferred_element_type=jnp.float32) \
        + o3b_ref[...]


def _head(u, bng_gamma, bng_beta, sega, segb, cnta, cntb,
          O1, o1, O2, o2, O3, o3):
    return pl.pallas_call(
        _head_body,
        out_shape=jax.ShapeDtypeStruct((G, OUT), jnp.float32),
    )(u, bng_gamma.reshape(1, GD), bng_beta.reshape(1, GD),
      sega, segb, cnta, cntb,
      O1, o1.reshape(1, BIGGER), O2, o2.reshape(1, BIGGER), O3,
      o3.reshape(1, OUT))


def kernel(x, u, batch, bn_gamma, bn_beta, bng_gamma, bng_beta,
           W1, b1, W2, b2, W3, b3, O1, o1, O2, o2, O3, o3):
    batchf = batch.astype(jnp.float32)
    xn, A = _prep(x, W1, bn_gamma, bn_beta)

    gid = jnp.arange(G, dtype=jnp.int32)
    counts = jnp.sum(batch[None, :] == gid[:, None], axis=1)
    ends = jnp.cumsum(counts)
    starts = ends - counts
    rb_first = batch[::RB]
    rb_last = batch[RB - 1::RB]
    span = ends[rb_last] - starts[rb_first]
    win_ok = jnp.all(span <= WIN - WB) & jnp.all(counts >= K)
    c0_blocks = jnp.clip(starts[rb_first] // WB, 0, (N - WIN) // WB)

    bcol = batchf.reshape(N, 1)
    brow = batchf.reshape(1, N)
    c0b = c0_blocks.astype(jnp.int32)
    idx = lax.cond(
        win_ok,
        lambda: jnp.concatenate([_topk_win(xn, bcol, brow, c0b, 0),
                                 _topk_win(xn, bcol, brow, c0b, 1)], axis=0),
        lambda: jnp.concatenate([_topk(xn, bcol, brow, 0),
                                 _topk(xn, bcol, brow, 1)], axis=0),
    )
    XJ_a = _sc_gather(xn, idx[:N // 2].reshape((N // 2) * K))
    XJ_b = _sc_gather(xn, idx[N // 2:].reshape((N // 2) * K))
    W1b = W1[D:, :]
    sega, cnta = _edge(A, XJ_a, bcol, W1b, W2, W3, b1, b2, b3, 0)
    segb, cntb = _edge(A, XJ_b, bcol, W1b, W2, W3, b1, b2, b3, 1)
    return _head(u, bng_gamma, bng_beta, sega, segb, cnta, cntb,
                 O1, o1, O2, o2, O3, o3)


# speedup vs baseline: 1.1936x; 1.1886x over previous
import functools

import jax
import jax.numpy as jnp
from jax import lax
from jax.experimental import pallas as pl
from jax.experimental.pallas import tpu as pltpu
from jax.experimental.pallas import tpu_sc as plsc

N, D, G, GD, BIG, BIGGER, OUT, K = 4096, 128, 16, 16, 256, 512, 1, 16
EPS = 1e-5

RB = 256
NB = 256
SC_WORKERS = 32
SC_CHUNK = 256


def _prep_body(x_ref, w1_ref, g_ref, b_ref, xn_ref, a_ref):
    x = x_ref[...]
    mu = jnp.mean(x, axis=0, keepdims=True)
    xc = x - mu
    var = jnp.mean(xc * xc, axis=0, keepdims=True)
    xn = xc / jnp.sqrt(var + EPS) * g_ref[...] + b_ref[...]
    xn_ref[...] = xn
    w1t = w1_ref[:D, :]
    w1b = w1_ref[D:, :]
    a_ref[...] = jnp.dot(xn, w1t - w1b, preferred_element_type=jnp.float32)


def _prep(x, W1, bn_gamma, bn_beta):
    return pl.pallas_call(
        _prep_body,
        out_shape=(
            jax.ShapeDtypeStruct((N, D), jnp.float32),
            jax.ShapeDtypeStruct((N, BIG), jnp.float32),
        ),
    )(x, W1, bn_gamma.reshape(1, D), bn_beta.reshape(1, D))


WIN = 1280
WB = 256


def _topk_win_body(c0_ref, xnb_ref, x0, x1, x2, x3, x4, bfb_ref,
                   r0, r1, r2, r3, r4, idx_ref):
    i = pl.program_id(0)
    c0 = c0_ref[i] * WB
    xnw = jnp.concatenate([x0[...], x1[...], x2[...], x3[...], x4[...]], axis=0)
    bfr = jnp.concatenate([r0[...], r1[...], r2[...], r3[...], r4[...]], axis=1)
    xnb = xnb_ref[...]
    sqb = jnp.sum(xnb * xnb, axis=1, keepdims=True)
    ysq = xnw * xnw
    sqr = jnp.dot(jnp.ones((1, D), jnp.float32), ysq.T,
                  preferred_element_type=jnp.float32)
    dist = sqb + sqr - 2.0 * jnp.dot(xnb, xnw.T, preferred_element_type=jnp.float32)
    cross = bfb_ref[...] != bfr
    d = jnp.maximum(jnp.where(cross, 1e10, dist), 0.0)
    iota = lax.broadcasted_iota(jnp.int32, (RB, WIN), 1)
    p = (lax.bitcast_convert_type(d, jnp.int32) & jnp.int32(-2048)) | iota
    big = jnp.int32(0x7FFFFFFF)
    for k in range(K):
        pm = jnp.min(p, axis=1, keepdims=True)
        idx_ref[:, k:k + 1] = (pm & 2047) + c0
        p = jnp.where(p == pm, big, p)


def _topk_win(xn, batchf_col, batchf_row, c0_blocks):
    grid_spec = pltpu.PrefetchScalarGridSpec(
        num_scalar_prefetch=1,
        grid=(N // RB,),
        in_specs=[
            pl.BlockSpec((RB, D), lambda i, c0: (i, 0)),
        ] + [
            pl.BlockSpec((WB, D), functools.partial(
                lambda j, i, c0: (c0[i] + j, 0), j)) for j in range(5)
        ] + [
            pl.BlockSpec((RB, 1), lambda i, c0: (i, 0)),
        ] + [
            pl.BlockSpec((1, WB), functools.partial(
                lambda j, i, c0: (0, c0[i] + j), j)) for j in range(5)
        ],
        out_specs=pl.BlockSpec((RB, K), lambda i, c0: (i, 0)),
    )
    return pl.pallas_call(
        _topk_win_body,
        grid_spec=grid_spec,
        out_shape=jax.ShapeDtypeStruct((N, K), jnp.int32),
    )(c0_blocks, xn, xn, xn, xn, xn, xn, batchf_col,
      batchf_row, batchf_row, batchf_row, batchf_row, batchf_row)


def _topk_body(xnb_ref, xn_ref, bfb_ref, bfr_ref, idx_ref):
    xnb = xnb_ref[...]
    xn = xn_ref[...]
    sqb = jnp.sum(xnb * xnb, axis=1, keepdims=True)
    ysq = xn * xn
    sqr = jnp.dot(jnp.ones((1, D), jnp.float32), ysq.T,
                  preferred_element_type=jnp.float32)
    dist = sqb + sqr - 2.0 * jnp.dot(xnb, xn.T, preferred_element_type=jnp.float32)
    cross = bfb_ref[...] != bfr_ref[...]
    d = jnp.where(cross, 1e10, dist)
    iota = lax.broadcasted_iota(jnp.int32, (RB, N), 1)
    inf = jnp.float32(jnp.inf)
    for k in range(K):
        m = jnp.min(d, axis=1, keepdims=True)
        amin = jnp.min(jnp.where(d == m, iota, N), axis=1, keepdims=True)
        idx_ref[:, k:k + 1] = amin
        d = jnp.where(iota == amin, inf, d)


def _topk(xn, batchf_col, batchf_row):
    return pl.pallas_call(
        _topk_body,
        grid=(N // RB,),
        in_specs=[
            pl.BlockSpec((RB, D), lambda i: (i, 0)),
            pl.BlockSpec((N, D), lambda i: (0, 0)),
            pl.BlockSpec((RB, 1), lambda i: (i, 0)),
            pl.BlockSpec((1, N), lambda i: (0, 0)),
        ],
        out_specs=pl.BlockSpec((RB, K), lambda i: (i, 0)),
        out_shape=jax.ShapeDtypeStruct((N, K), jnp.int32),
    )(xn, xn, batchf_col, batchf_row)


def _sc_gather_body(nrows, table_hbm, idx_hbm, out_hbm, idx_v, rows_v, sem):
    wid = lax.axis_index("s") * 2 + lax.axis_index("c")
    per_w = nrows // SC_WORKERS
    base = wid * per_w

    def chunk(c, _):
        off = pl.multiple_of(base + c * SC_CHUNK, SC_CHUNK)
        pltpu.sync_copy(idx_hbm.at[pl.ds(off, SC_CHUNK)], idx_v)
        pltpu.async_copy(table_hbm.at[idx_v], rows_v, sem).wait()
        pltpu.sync_copy(rows_v, out_hbm.at[pl.ds(off, SC_CHUNK)])
        return _

    lax.fori_loop(0, per_w // SC_CHUNK, chunk, None)


def _sc_gather(table, idx_flat):
    nrows = idx_flat.shape[0]
    mesh = plsc.VectorSubcoreMesh(core_axis_name="c", subcore_axis_name="s")
    kfn = functools.partial(
        pl.kernel,
        mesh=mesh,
        out_type=jax.ShapeDtypeStruct((nrows, D), jnp.float32),
        scratch_types=[
            pltpu.VMEM((SC_CHUNK,), jnp.int32),
            pltpu.VMEM((SC_CHUNK, D), jnp.float32),
            pltpu.SemaphoreType.DMA,
        ],
    )(functools.partial(_sc_gather_body, nrows))
    return kfn(table, idx_flat)


def _edge_body(a_ref, xj_ref, bfb_ref, w1b_ref, w2_ref, w3_ref,
               b1_ref, b2_ref, b3_ref, seg_ref, cnt_ref):
    @pl.when(pl.program_id(0) == 0)
    def _init():
        seg_ref[...] = jnp.zeros_like(seg_ref)
        cnt_ref[...] = jnp.zeros_like(cnt_ref)

    a = a_ref[...]
    a_rep = jnp.broadcast_to(a[:, None, :], (NB, K, BIG)).reshape(NB * K, BIG)
    bj = jnp.dot(xj_ref[...], w1b_ref[...], preferred_element_type=jnp.float32)
    h1 = jnp.maximum(a_rep + bj + b1_ref[...], 0.0)
    h2 = jnp.maximum(jnp.dot(h1, w2_ref[...], preferred_element_type=jnp.float32)
                     + b2_ref[...], 0.0)
    h3 = jnp.maximum(jnp.dot(h2, w3_ref[...], preferred_element_type=jnp.float32)
                     + b3_ref[...], 0.0)
    hx = jnp.mean(h3.reshape(NB, K, BIG), axis=1)
    gids = lax.broadcasted_iota(jnp.int32, (1, G), 1).astype(jnp.float32)
    onehot = (bfb_ref[...] == gids).astype(jnp.float32)
    seg_ref[...] += lax.dot_general(onehot, hx, (((0,), (0,)), ((), ())),
                                    preferred_element_type=jnp.float32)
    cnt_ref[...] += jnp.broadcast_to(
        jnp.sum(onehot, axis=0)[:, None], (G, BIG))


def _edge(A, XJ, batchf_col, W1b, W2, W3, b1, b2, b3):
    return pl.pallas_call(
        _edge_body,
        grid=(N // NB,),
        in_specs=[
            pl.BlockSpec((NB, BIG), lambda i: (i, 0)),
            pl.BlockSpec((NB * K, D), lambda i: (i, 0)),
            pl.BlockSpec((NB, 1), lambda i: (i, 0)),
            pl.BlockSpec((D, BIG), lambda i: (0, 0)),
            pl.BlockSpec((BIG, BIG), lambda i: (0, 0)),
            pl.BlockSpec((BIG, BIG), lambda i: (0, 0)),
            pl.BlockSpec((1, BIG), lambda i: (0, 0)),
            pl.BlockSpec((1, BIG), lambda i: (0, 0)),
            pl.BlockSpec((1, BIG), lambda i: (0, 0)),
        ],
        out_specs=(
            pl.BlockSpec((G, BIG), lambda i: (0, 0)),
            pl.BlockSpec((G, BIG), lambda i: (0, 0)),
        ),
        out_shape=(
            jax.ShapeDtypeStruct((G, BIG), jnp.float32),
            jax.ShapeDtypeStruct((G, BIG), jnp.float32),
        ),
    )(A, XJ, batchf_col, W1b, W2, W3,
      b1.reshape(1, BIG), b2.reshape(1, BIG), b3.reshape(1, BIG))


def _head_body(u_ref, gg_ref, gb_ref, seg_ref, cnt_ref,
               o1w_ref, o1b_ref, o2w_ref, o2b_ref, o3w_ref, o3b_ref, out_ref):
    u = u_ref[...]
    mu = jnp.mean(u, axis=0, keepdims=True)
    uc = u - mu
    var = jnp.mean(uc * uc, axis=0, keepdims=True)
    u1 = uc / jnp.sqrt(var + EPS) * gg_ref[...] + gb_ref[...]
    u2 = seg_ref[...] / jnp.maximum(cnt_ref[...], 1.0)
    uu = jnp.concatenate([u1, u2], axis=1)
    o = jnp.maximum(jnp.dot(uu, o1w_ref[...], preferred_element_type=jnp.float32)
                    + o1b_ref[...], 0.0)
    o = jnp.maximum(jnp.dot(o, o2w_ref[...], preferred_element_type=jnp.float32)
                    + o2b_ref[...], 0.0)
    out_ref[...] = jnp.dot(o, o3w_ref[...], preferred_element_type=jnp.float32) \
        + o3b_ref[...]


def _head(u, bng_gamma, bng_beta, seg, cnt, O1, o1, O2, o2, O3, o3):
    return pl.pallas_call(
        _head_body,
        out_shape=jax.ShapeDtypeStruct((G, OUT), jnp.float32),
    )(u, bng_gamma.reshape(1, GD), bng_beta.reshape(1, GD), seg, cnt,
      O1, o1.reshape(1, BIGGER), O2, o2.reshape(1, BIGGER), O3,
      o3.reshape(1, OUT))


def kernel(x, u, batch, bn_gamma, bn_beta, bng_gamma, bng_beta,
           W1, b1, W2, b2, W3, b3, O1, o1, O2, o2, O3, o3):
    batchf = batch.astype(jnp.float32)
    xn, A = _prep(x, W1, bn_gamma, bn_beta)

    gid = jnp.arange(G, dtype=jnp.int32)
    counts = jnp.sum(batch[None, :] == gid[:, None], axis=1)
    ends = jnp.cumsum(counts)
    starts = ends - counts
    rb_first = batch[::RB]
    rb_last = batch[RB - 1::RB]
    span = ends[rb_last] - starts[rb_first]
    win_ok = jnp.all(span <= WIN - WB) & jnp.all(counts >= K)
    c0_blocks = jnp.clip(starts[rb_first] // WB, 0, (N - WIN) // WB)

    bcol = batchf.reshape(N, 1)
    brow = batchf.reshape(1, N)
    c0b = c0_blocks.astype(jnp.int32)
    idx = lax.cond(
        win_ok,
        lambda: _topk_win(xn, bcol, brow, c0b),
        lambda: _topk(xn, bcol, brow),
    )
    XJ = _sc_gather(xn, idx.reshape(N * K))
    seg, cnt = _edge(A, XJ, bcol, W1[D:, :], W2, W3, b1, b2, b3)
    return _head(u, bng_gamma, bng_beta, seg, cnt, O1, o1, O2, o2, O3, o3)


# speedup vs baseline: 1.1968x; 1.0027x over previous
import functools

import jax
import jax.numpy as jnp
from jax import lax
from jax.experimental import pallas as pl
from jax.experimental.pallas import tpu as pltpu
from jax.experimental.pallas import tpu_sc as plsc

N, D, G, GD, BIG, BIGGER, OUT, K = 4096, 128, 16, 16, 256, 512, 1, 16
EPS = 1e-5

RB = 256
NB = 256
SC_WORKERS = 32
SC_CHUNK = 256


def _prep_body(x_ref, w1_ref, g_ref, b_ref, xn_ref, a_ref):
    x = x_ref[...]
    mu = jnp.mean(x, axis=0, keepdims=True)
    xc = x - mu
    var = jnp.mean(xc * xc, axis=0, keepdims=True)
    xn = xc / jnp.sqrt(var + EPS) * g_ref[...] + b_ref[...]
    xn_ref[...] = xn
    w1t = w1_ref[:D, :]
    w1b = w1_ref[D:, :]
    a_ref[...] = jnp.dot(xn, w1t - w1b, preferred_element_type=jnp.float32)


def _prep(x, W1, bn_gamma, bn_beta):
    return pl.pallas_call(
        _prep_body,
        out_shape=(
            jax.ShapeDtypeStruct((N, D), jnp.float32),
            jax.ShapeDtypeStruct((N, BIG), jnp.float32),
        ),
    )(x, W1, bn_gamma.reshape(1, D), bn_beta.reshape(1, D))


WIN = 1280
WB = 256


def _topk_win_body(c0_ref, xnb_ref, x0, x1, x2, x3, x4, bfb_ref,
                   r0, r1, r2, r3, r4, idx_ref):
    i = pl.program_id(0)
    c0 = c0_ref[i] * WB
    xnw = jnp.concatenate([x0[...], x1[...], x2[...], x3[...], x4[...]], axis=0)
    bfr = jnp.concatenate([r0[...], r1[...], r2[...], r3[...], r4[...]], axis=1)
    xnb = xnb_ref[...]
    sqb = jnp.sum(xnb * xnb, axis=1, keepdims=True)
    ysq = xnw * xnw
    sqr = jnp.dot(jnp.ones((1, D), jnp.float32), ysq.T,
                  preferred_element_type=jnp.float32)
    dist = sqb + sqr - 2.0 * jnp.dot(xnb, xnw.T, preferred_element_type=jnp.float32)
    cross = bfb_ref[...] != bfr
    d = jnp.maximum(jnp.where(cross, 1e10, dist), 0.0)
    iota = lax.broadcasted_iota(jnp.int32, (RB, WIN), 1)
    p = (lax.bitcast_convert_type(d, jnp.int32) & jnp.int32(-2048)) | iota
    big = jnp.int32(0x7FFFFFFF)
    pm = jnp.min(p, axis=1, keepdims=True)
    idx_ref[:, 0:1] = (pm & 2047) + c0
    for k in range(1, K):
        pm = jnp.min(jnp.where(p > pm, p, big), axis=1, keepdims=True)
        idx_ref[:, k:k + 1] = (pm & 2047) + c0


def _topk_win(xn, batchf_col, batchf_row, c0_blocks):
    grid_spec = pltpu.PrefetchScalarGridSpec(
        num_scalar_prefetch=1,
        grid=(N // RB,),
        in_specs=[
            pl.BlockSpec((RB, D), lambda i, c0: (i, 0)),
        ] + [
            pl.BlockSpec((WB, D), functools.partial(
                lambda j, i, c0: (c0[i] + j, 0), j)) for j in range(5)
        ] + [
            pl.BlockSpec((RB, 1), lambda i, c0: (i, 0)),
        ] + [
            pl.BlockSpec((1, WB), functools.partial(
                lambda j, i, c0: (0, c0[i] + j), j)) for j in range(5)
        ],
        out_specs=pl.BlockSpec((RB, K), lambda i, c0: (i, 0)),
    )
    return pl.pallas_call(
        _topk_win_body,
        grid_spec=grid_spec,
        out_shape=jax.ShapeDtypeStruct((N, K), jnp.int32),
    )(c0_blocks, xn, xn, xn, xn, xn, xn, batchf_col,
      batchf_row, batchf_row, batchf_row, batchf_row, batchf_row)


def _topk_body(xnb_ref, xn_ref, bfb_ref, bfr_ref, idx_ref):
    xnb = xnb_ref[...]
    xn = xn_ref[...]
    sqb = jnp.sum(xnb * xnb, axis=1, keepdims=True)
    ysq = xn * xn
    sqr = jnp.dot(jnp.ones((1, D), jnp.float32), ysq.T,
                  preferred_element_type=jnp.float32)
    dist = sqb + sqr - 2.0 * jnp.dot(xnb, xn.T, preferred_element_type=jnp.float32)
    cross = bfb_ref[...] != bfr_ref[...]
    d = jnp.where(cross, 1e10, dist)
    iota = lax.broadcasted_iota(jnp.int32, (RB, N), 1)
    inf = jnp.float32(jnp.inf)
    for k in range(K):
        m = jnp.min(d, axis=1, keepdims=True)
        amin = jnp.min(jnp.where(d == m, iota, N), axis=1, keepdims=True)
        idx_ref[:, k:k + 1] = amin
        d = jnp.where(iota == amin, inf, d)


def _topk(xn, batchf_col, batchf_row):
    return pl.pallas_call(
        _topk_body,
        grid=(N // RB,),
        in_specs=[
            pl.BlockSpec((RB, D), lambda i: (i, 0)),
            pl.BlockSpec((N, D), lambda i: (0, 0)),
            pl.BlockSpec((RB, 1), lambda i: (i, 0)),
            pl.BlockSpec((1, N), lambda i: (0, 0)),
        ],
        out_specs=pl.BlockSpec((RB, K), lambda i: (i, 0)),
        out_shape=jax.ShapeDtypeStruct((N, K), jnp.int32),
    )(xn, xn, batchf_col, batchf_row)


def _sc_gather_body(nrows, table_hbm, idx_hbm, out_hbm, idx_v, rows_v, sem):
    wid = lax.axis_index("s") * 2 + lax.axis_index("c")
    per_w = nrows // SC_WORKERS
    base = wid * per_w

    def chunk(c, _):
        off = pl.multiple_of(base + c * SC_CHUNK, SC_CHUNK)
        pltpu.sync_copy(idx_hbm.at[pl.ds(off, SC_CHUNK)], idx_v)
        pltpu.async_copy(table_hbm.at[idx_v], rows_v, sem).wait()
        pltpu.sync_copy(rows_v, out_hbm.at[pl.ds(off, SC_CHUNK)])
        return _

    lax.fori_loop(0, per_w // SC_CHUNK, chunk, None)


def _sc_gather(table, idx_flat):
    nrows = idx_flat.shape[0]
    mesh = plsc.VectorSubcoreMesh(core_axis_name="c", subcore_axis_name="s")
    kfn = functools.partial(
        pl.kernel,
        mesh=mesh,
        out_type=jax.ShapeDtypeStruct((nrows, D), jnp.float32),
        scratch_types=[
            pltpu.VMEM((SC_CHUNK,), jnp.int32),
            pltpu.VMEM((SC_CHUNK, D), jnp.float32),
            pltpu.SemaphoreType.DMA,
        ],
    )(functools.partial(_sc_gather_body, nrows))
    return kfn(table, idx_flat)


def _edge_body(a_ref, xj_ref, bfb_ref, w1b_ref, w2_ref, w3_ref,
               b1_ref, b2_ref, b3_ref, seg_ref, cnt_ref):
    @pl.when(pl.program_id(0) == 0)
    def _init():
        seg_ref[...] = jnp.zeros_like(seg_ref)
        cnt_ref[...] = jnp.zeros_like(cnt_ref)

    a = a_ref[...]
    a_rep = jnp.broadcast_to(a[:, None, :], (NB, K, BIG)).reshape(NB * K, BIG)
    bj = jnp.dot(xj_ref[...], w1b_ref[...], preferred_element_type=jnp.float32)
    h1 = jnp.maximum(a_rep + bj + b1_ref[...], 0.0)
    h2 = jnp.maximum(jnp.dot(h1, w2_ref[...], preferred_element_type=jnp.float32)
                     + b2_ref[...], 0.0)
    h3 = jnp.maximum(jnp.dot(h2, w3_ref[...], preferred_element_type=jnp.float32)
                     + b3_ref[...], 0.0)
    hx = jnp.mean(h3.reshape(NB, K, BIG), axis=1)
    gids = lax.broadcasted_iota(jnp.int32, (1, G), 1).astype(jnp.float32)
    onehot = (bfb_ref[...] == gids).astype(jnp.float32)
    seg_ref[...] += lax.dot_general(onehot, hx, (((0,), (0,)), ((), ())),
                                    preferred_element_type=jnp.float32)
    cnt_ref[...] += jnp.broadcast_to(
        jnp.sum(onehot, axis=0)[:, None], (G, BIG))


def _edge(A, XJ, batchf_col, W1b, W2, W3, b1, b2, b3):
    return pl.pallas_call(
        _edge_body,
        grid=(N // NB,),
        in_specs=[
            pl.BlockSpec((NB, BIG), lambda i: (i, 0)),
            pl.BlockSpec((NB * K, D), lambda i: (i, 0)),
            pl.BlockSpec((NB, 1), lambda i: (i, 0)),
            pl.BlockSpec((D, BIG), lambda i: (0, 0)),
            pl.BlockSpec((BIG, BIG), lambda i: (0, 0)),
            pl.BlockSpec((BIG, BIG), lambda i: (0, 0)),
            pl.BlockSpec((1, BIG), lambda i: (0, 0)),
            pl.BlockSpec((1, BIG), lambda i: (0, 0)),
            pl.BlockSpec((1, BIG), lambda i: (0, 0)),
        ],
        out_specs=(
            pl.BlockSpec((G, BIG), lambda i: (0, 0)),
            pl.BlockSpec((G, BIG), lambda i: (0, 0)),
        ),
        out_shape=(
            jax.ShapeDtypeStruct((G, BIG), jnp.float32),
            jax.ShapeDtypeStruct((G, BIG), jnp.float32),
        ),
    )(A, XJ, batchf_col, W1b, W2, W3,
      b1.reshape(1, BIG), b2.reshape(1, BIG), b3.reshape(1, BIG))


def _head_body(u_ref, gg_ref, gb_ref, seg_ref, cnt_ref,
               o1w_ref, o1b_ref, o2w_ref, o2b_ref, o3w_ref, o3b_ref, out_ref):
    u = u_ref[...]
    mu = jnp.mean(u, axis=0, keepdims=True)
    uc = u - mu
    var = jnp.mean(uc * uc, axis=0, keepdims=True)
    u1 = uc / jnp.sqrt(var + EPS) * gg_ref[...] + gb_ref[...]
    u2 = seg_ref[...] / jnp.maximum(cnt_ref[...], 1.0)
    uu = jnp.concatenate([u1, u2], axis=1)
    o = jnp.maximum(jnp.dot(uu, o1w_ref[...], preferred_element_type=jnp.float32)
                    + o1b_ref[...], 0.0)
    o = jnp.maximum(jnp.dot(o, o2w_ref[...], preferred_element_type=jnp.float32)
                    + o2b_ref[...], 0.0)
    out_ref[...] = jnp.dot(o, o3w_ref[...], preferred_element_type=jnp.float32) \
        + o3b_ref[...]


def _head(u, bng_gamma, bng_beta, seg, cnt, O1, o1, O2, o2, O3, o3):
    return pl.pallas_call(
        _head_body,
        out_shape=jax.ShapeDtypeStruct((G, OUT), jnp.float32),
    )(u, bng_gamma.reshape(1, GD), bng_beta.reshape(1, GD), seg, cnt,
      O1, o1.reshape(1, BIGGER), O2, o2.reshape(1, BIGGER), O3,
      o3.reshape(1, OUT))


def kernel(x, u, batch, bn_gamma, bn_beta, bng_gamma, bng_beta,
           W1, b1, W2, b2, W3, b3, O1, o1, O2, o2, O3, o3):
    batchf = batch.astype(jnp.float32)
    xn, A = _prep(x, W1, bn_gamma, bn_beta)

    gid = jnp.arange(G, dtype=jnp.int32)
    counts = jnp.sum(batch[None, :] == gid[:, None], axis=1)
    ends = jnp.cumsum(counts)
    starts = ends - counts
    rb_first = batch[::RB]
    rb_last = batch[RB - 1::RB]
    span = ends[rb_last] - starts[rb_first]
    win_ok = jnp.all(span <= WIN - WB) & jnp.all(counts >= K)
    c0_blocks = jnp.clip(starts[rb_first] // WB, 0, (N - WIN) // WB)

    bcol = batchf.reshape(N, 1)
    brow = batchf.reshape(1, N)
    c0b = c0_blocks.astype(jnp.int32)
    idx = lax.cond(
        win_ok,
        lambda: _topk_win(xn, bcol, brow, c0b),
        lambda: _topk(xn, bcol, brow),
    )
    XJ = _sc_gather(xn, idx.reshape(N * K))
    seg, cnt = _edge(A, XJ, bcol, W1[D:, :], W2, W3, b1, b2, b3)
    return _head(u, bng_gamma, bng_beta, seg, cnt, O1, o1, O2, o2, O3, o3)


# speedup vs baseline: 1.2055x; 1.0073x over previous
import functools

import jax
import jax.numpy as jnp
from jax import lax
from jax.experimental import pallas as pl
from jax.experimental.pallas import tpu as pltpu
from jax.experimental.pallas import tpu_sc as plsc

N, D, G, GD, BIG, BIGGER, OUT, K = 4096, 128, 16, 16, 256, 512, 1, 16
EPS = 1e-5

RB = 256
NB = 512
SC_WORKERS = 32
SC_CHUNK = 256


def _prep_body(x_ref, w1_ref, g_ref, b_ref, xn_ref, a_ref):
    x = x_ref[...]
    mu = jnp.mean(x, axis=0, keepdims=True)
    xc = x - mu
    var = jnp.mean(xc * xc, axis=0, keepdims=True)
    xn = xc / jnp.sqrt(var + EPS) * g_ref[...] + b_ref[...]
    xn_ref[...] = xn
    w1t = w1_ref[:D, :]
    w1b = w1_ref[D:, :]
    a_ref[...] = jnp.dot(xn, w1t - w1b, preferred_element_type=jnp.float32)


def _prep(x, W1, bn_gamma, bn_beta):
    return pl.pallas_call(
        _prep_body,
        out_shape=(
            jax.ShapeDtypeStruct((N, D), jnp.float32),
            jax.ShapeDtypeStruct((N, BIG), jnp.float32),
        ),
    )(x, W1, bn_gamma.reshape(1, D), bn_beta.reshape(1, D))


WIN = 1280
WB = 256


def _topk_win_body(c0_ref, xnb_ref, x0, x1, x2, x3, x4, bfb_ref,
                   r0, r1, r2, r3, r4, idx_ref):
    i = pl.program_id(0)
    c0 = c0_ref[i] * WB
    xnw = jnp.concatenate([x0[...], x1[...], x2[...], x3[...], x4[...]], axis=0)
    bfr = jnp.concatenate([r0[...], r1[...], r2[...], r3[...], r4[...]], axis=1)
    xnb = xnb_ref[...]
    sqb = jnp.sum(xnb * xnb, axis=1, keepdims=True)
    ysq = xnw * xnw
    sqr = jnp.dot(jnp.ones((1, D), jnp.float32), ysq.T,
                  preferred_element_type=jnp.float32)
    dist = sqb + sqr - 2.0 * jnp.dot(xnb, xnw.T, preferred_element_type=jnp.float32)
    cross = bfb_ref[...] != bfr
    d = jnp.maximum(jnp.where(cross, 1e10, dist), 0.0)
    iota = lax.broadcasted_iota(jnp.int32, (RB, WIN), 1)
    p = (lax.bitcast_convert_type(d, jnp.int32) & jnp.int32(-2048)) | iota
    big = jnp.int32(0x7FFFFFFF)
    pm = jnp.min(p, axis=1, keepdims=True)
    idx_ref[:, 0:1] = (pm & 2047) + c0
    for k in range(1, K):
        pm = jnp.min(jnp.where(p > pm, p, big), axis=1, keepdims=True)
        idx_ref[:, k:k + 1] = (pm & 2047) + c0


def _topk_win(xn, batchf_col, batchf_row, c0_blocks):
    grid_spec = pltpu.PrefetchScalarGridSpec(
        num_scalar_prefetch=1,
        grid=(N // RB,),
        in_specs=[
            pl.BlockSpec((RB, D), lambda i, c0: (i, 0)),
        ] + [
            pl.BlockSpec((WB, D), functools.partial(
                lambda j, i, c0: (c0[i] + j, 0), j)) for j in range(5)
        ] + [
            pl.BlockSpec((RB, 1), lambda i, c0: (i, 0)),
        ] + [
            pl.BlockSpec((1, WB), functools.partial(
                lambda j, i, c0: (0, c0[i] + j), j)) for j in range(5)
        ],
        out_specs=pl.BlockSpec((RB, K), lambda i, c0: (i, 0)),
    )
    return pl.pallas_call(
        _topk_win_body,
        grid_spec=grid_spec,
        out_shape=jax.ShapeDtypeStruct((N, K), jnp.int32),
    )(c0_blocks, xn, xn, xn, xn, xn, xn, batchf_col,
      batchf_row, batchf_row, batchf_row, batchf_row, batchf_row)


def _topk_body(xnb_ref, xn_ref, bfb_ref, bfr_ref, idx_ref):
    xnb = xnb_ref[...]
    xn = xn_ref[...]
    sqb = jnp.sum(xnb * xnb, axis=1, keepdims=True)
    ysq = xn * xn
    sqr = jnp.dot(jnp.ones((1, D), jnp.float32), ysq.T,
                  preferred_element_type=jnp.float32)
    dist = sqb + sqr - 2.0 * jnp.dot(xnb, xn.T, preferred_element_type=jnp.float32)
    cross = bfb_ref[...] != bfr_ref[...]
    d = jnp.where(cross, 1e10, dist)
    iota = lax.broadcasted_iota(jnp.int32, (RB, N), 1)
    inf = jnp.float32(jnp.inf)
    for k in range(K):
        m = jnp.min(d, axis=1, keepdims=True)
        amin = jnp.min(jnp.where(d == m, iota, N), axis=1, keepdims=True)
        idx_ref[:, k:k + 1] = amin
        d = jnp.where(iota == amin, inf, d)


def _topk(xn, batchf_col, batchf_row):
    return pl.pallas_call(
        _topk_body,
        grid=(N // RB,),
        in_specs=[
            pl.BlockSpec((RB, D), lambda i: (i, 0)),
            pl.BlockSpec((N, D), lambda i: (0, 0)),
            pl.BlockSpec((RB, 1), lambda i: (i, 0)),
            pl.BlockSpec((1, N), lambda i: (0, 0)),
        ],
        out_specs=pl.BlockSpec((RB, K), lambda i: (i, 0)),
        out_shape=jax.ShapeDtypeStruct((N, K), jnp.int32),
    )(xn, xn, batchf_col, batchf_row)


def _sc_gather_body(nrows, table_hbm, idx_hbm, out_hbm, idx_v, rows_v, sem):
    wid = lax.axis_index("s") * 2 + lax.axis_index("c")
    per_w = nrows // SC_WORKERS
    base = wid * per_w

    def chunk(c, _):
        off = pl.multiple_of(base + c * SC_CHUNK, SC_CHUNK)
        pltpu.sync_copy(idx_hbm.at[pl.ds(off, SC_CHUNK)], idx_v)
        pltpu.async_copy(table_hbm.at[idx_v], rows_v, sem).wait()
        pltpu.sync_copy(rows_v, out_hbm.at[pl.ds(off, SC_CHUNK)])
        return _

    lax.fori_loop(0, per_w // SC_CHUNK, chunk, None)


def _sc_gather(table, idx_flat):
    nrows = idx_flat.shape[0]
    mesh = plsc.VectorSubcoreMesh(core_axis_name="c", subcore_axis_name="s")
    kfn = functools.partial(
        pl.kernel,
        mesh=mesh,
        out_type=jax.ShapeDtypeStruct((nrows, D), jnp.float32),
        scratch_types=[
            pltpu.VMEM((SC_CHUNK,), jnp.int32),
            pltpu.VMEM((SC_CHUNK, D), jnp.float32),
            pltpu.SemaphoreType.DMA,
        ],
    )(functools.partial(_sc_gather_body, nrows))
    return kfn(table, idx_flat)


def _edge_body(a_ref, xj_ref, bfb_ref, w1b_ref, w2_ref, w3_ref,
               b1_ref, b2_ref, b3_ref, seg_ref, cnt_ref):
    @pl.when(pl.program_id(0) == 0)
    def _init():
        seg_ref[...] = jnp.zeros_like(seg_ref)
        cnt_ref[...] = jnp.zeros_like(cnt_ref)

    a = a_ref[...]
    a_rep = jnp.broadcast_to(a[:, None, :], (NB, K, BIG)).reshape(NB * K, BIG)
    bj = jnp.dot(xj_ref[...], w1b_ref[...], preferred_element_type=jnp.float32)
    h1 = jnp.maximum(a_rep + bj + b1_ref[...], 0.0)
    h2 = jnp.maximum(jnp.dot(h1, w2_ref[...], preferred_element_type=jnp.float32)
                     + b2_ref[...], 0.0)
    h3 = jnp.maximum(jnp.dot(h2, w3_ref[...], preferred_element_type=jnp.float32)
                     + b3_ref[...], 0.0)
    hx = jnp.mean(h3.reshape(NB, K, BIG), axis=1)
    gids = lax.broadcasted_iota(jnp.int32, (1, G), 1).astype(jnp.float32)
    onehot = (bfb_ref[...] == gids).astype(jnp.float32)
    seg_ref[...] += lax.dot_general(onehot, hx, (((0,), (0,)), ((), ())),
                                    preferred_element_type=jnp.float32)
    cnt_ref[...] += jnp.broadcast_to(
        jnp.sum(onehot, axis=0)[:, None], (G, BIG))


def _edge(A, XJ, batchf_col, W1b, W2, W3, b1, b2, b3):
    return pl.pallas_call(
        _edge_body,
        grid=(N // NB,),
        in_specs=[
            pl.BlockSpec((NB, BIG), lambda i: (i, 0)),
            pl.BlockSpec((NB * K, D), lambda i: (i, 0)),
            pl.BlockSpec((NB, 1), lambda i: (i, 0)),
            pl.BlockSpec((D, BIG), lambda i: (0, 0)),
            pl.BlockSpec((BIG, BIG), lambda i: (0, 0)),
            pl.BlockSpec((BIG, BIG), lambda i: (0, 0)),
            pl.BlockSpec((1, BIG), lambda i: (0, 0)),
            pl.BlockSpec((1, BIG), lambda i: (0, 0)),
            pl.BlockSpec((1, BIG), lambda i: (0, 0)),
        ],
        out_specs=(
            pl.BlockSpec((G, BIG), lambda i: (0, 0)),
            pl.BlockSpec((G, BIG), lambda i: (0, 0)),
        ),
        out_shape=(
            jax.ShapeDtypeStruct((G, BIG), jnp.float32),
            jax.ShapeDtypeStruct((G, BIG), jnp.float32),
        ),
    )(A, XJ, batchf_col, W1b, W2, W3,
      b1.reshape(1, BIG), b2.reshape(1, BIG), b3.reshape(1, BIG))


def _head_body(u_ref, gg_ref, gb_ref, seg_ref, cnt_ref,
               o1w_ref, o1b_ref, o2w_ref, o2b_ref, o3w_ref, o3b_ref, out_ref):
    u = u_ref[...]
    mu = jnp.mean(u, axis=0, keepdims=True)
    uc = u - mu
    var = jnp.mean(uc * uc, axis=0, keepdims=True)
    u1 = uc / jnp.sqrt(var + EPS) * gg_ref[...] + gb_ref[...]
    u2 = seg_ref[...] / jnp.maximum(cnt_ref[...], 1.0)
    uu = jnp.concatenate([u1, u2], axis=1)
    o = jnp.maximum(jnp.dot(uu, o1w_ref[...], preferred_element_type=jnp.float32)
                    + o1b_ref[...], 0.0)
    o = jnp.maximum(jnp.dot(o, o2w_ref[...], preferred_element_type=jnp.float32)
                    + o2b_ref[...], 0.0)
    out_ref[...] = jnp.dot(o, o3w_ref[...], preferred_element_type=jnp.float32) \
        + o3b_ref[...]


def _head(u, bng_gamma, bng_beta, seg, cnt, O1, o1, O2, o2, O3, o3):
    return pl.pallas_call(
        _head_body,
        out_shape=jax.ShapeDtypeStruct((G, OUT), jnp.float32),
    )(u, bng_gamma.reshape(1, GD), bng_beta.reshape(1, GD), seg, cnt,
      O1, o1.reshape(1, BIGGER), O2, o2.reshape(1, BIGGER), O3,
      o3.reshape(1, OUT))


def kernel(x, u, batch, bn_gamma, bn_beta, bng_gamma, bng_beta,
           W1, b1, W2, b2, W3, b3, O1, o1, O2, o2, O3, o3):
    batchf = batch.astype(jnp.float32)
    xn, A = _prep(x, W1, bn_gamma, bn_beta)

    gid = jnp.arange(G, dtype=jnp.int32)
    counts = jnp.sum(batch[None, :] == gid[:, None], axis=1)
    ends = jnp.cumsum(counts)
    starts = ends - counts
    rb_first = batch[::RB]
    rb_last = batch[RB - 1::RB]
    span = ends[rb_last] - starts[rb_first]
    win_ok = jnp.all(span <= WIN - WB) & jnp.all(counts >= K)
    c0_blocks = jnp.clip(starts[rb_first] // WB, 0, (N - WIN) // WB)

    bcol = batchf.reshape(N, 1)
    brow = batchf.reshape(1, N)
    c0b = c0_blocks.astype(jnp.int32)
    idx = lax.cond(
        win_ok,
        lambda: _topk_win(xn, bcol, brow, c0b),
        lambda: _topk(xn, bcol, brow),
    )
    XJ = _sc_gather(xn, idx.reshape(N * K))
    seg, cnt = _edge(A, XJ, bcol, W1[D:, :], W2, W3, b1, b2, b3)
    return _head(u, bng_gamma, bng_beta, seg, cnt, O1, o1, O2, o2, O3, o3)


# speedup vs baseline: 1.2213x; 1.0131x over previous
import functools

import jax
import jax.numpy as jnp
from jax import lax
from jax.experimental import pallas as pl
from jax.experimental.pallas import tpu as pltpu
from jax.experimental.pallas import tpu_sc as plsc

N, D, G, GD, BIG, BIGGER, OUT, K = 4096, 128, 16, 16, 256, 512, 1, 16
EPS = 1e-5

RB = 256
NB = 512
SC_WORKERS = 32
SC_CHUNK = 512


def _prep_body(x_ref, w1_ref, g_ref, b_ref, xn_ref, a_ref):
    x = x_ref[...]
    mu = jnp.mean(x, axis=0, keepdims=True)
    xc = x - mu
    var = jnp.mean(xc * xc, axis=0, keepdims=True)
    xn = xc / jnp.sqrt(var + EPS) * g_ref[...] + b_ref[...]
    xn_ref[...] = xn
    w1t = w1_ref[:D, :]
    w1b = w1_ref[D:, :]
    a_ref[...] = jnp.dot(xn, w1t - w1b, preferred_element_type=jnp.float32)


def _prep(x, W1, bn_gamma, bn_beta):
    return pl.pallas_call(
        _prep_body,
        out_shape=(
            jax.ShapeDtypeStruct((N, D), jnp.float32),
            jax.ShapeDtypeStruct((N, BIG), jnp.float32),
        ),
    )(x, W1, bn_gamma.reshape(1, D), bn_beta.reshape(1, D))


WIN = 1280
WB = 256


def _topk_win_body(c0_ref, xnb_ref, x0, x1, x2, x3, x4, bfb_ref,
                   r0, r1, r2, r3, r4, idx_ref):
    i = pl.program_id(0)
    c0 = c0_ref[i] * WB
    xnw = jnp.concatenate([x0[...], x1[...], x2[...], x3[...], x4[...]], axis=0)
    bfr = jnp.concatenate([r0[...], r1[...], r2[...], r3[...], r4[...]], axis=1)
    xnb = xnb_ref[...]
    sqb = jnp.sum(xnb * xnb, axis=1, keepdims=True)
    ysq = xnw * xnw
    sqr = jnp.dot(jnp.ones((1, D), jnp.float32), ysq.T,
                  preferred_element_type=jnp.float32)
    dist = sqb + sqr - 2.0 * jnp.dot(xnb, xnw.T, preferred_element_type=jnp.float32)
    cross = bfb_ref[...] != bfr
    d = jnp.maximum(jnp.where(cross, 1e10, dist), 0.0)
    iota = lax.broadcasted_iota(jnp.int32, (RB, WIN), 1)
    p = (lax.bitcast_convert_type(d, jnp.int32) & jnp.int32(-2048)) | iota
    big = jnp.int32(0x7FFFFFFF)
    pm = jnp.min(p, axis=1, keepdims=True)
    idx_ref[:, 0:1] = (pm & 2047) + c0
    for k in range(1, K):
        pm = jnp.min(jnp.where(p > pm, p, big), axis=1, keepdims=True)
        idx_ref[:, k:k + 1] = (pm & 2047) + c0


def _topk_win(xn, batchf_col, batchf_row, c0_blocks):
    grid_spec = pltpu.PrefetchScalarGridSpec(
        num_scalar_prefetch=1,
        grid=(N // RB,),
        in_specs=[
            pl.BlockSpec((RB, D), lambda i, c0: (i, 0)),
        ] + [
            pl.BlockSpec((WB, D), functools.partial(
                lambda j, i, c0: (c0[i] + j, 0), j)) for j in range(5)
        ] + [
            pl.BlockSpec((RB, 1), lambda i, c0: (i, 0)),
        ] + [
            pl.BlockSpec((1, WB), functools.partial(
                lambda j, i, c0: (0, c0[i] + j), j)) for j in range(5)
        ],
        out_specs=pl.BlockSpec((RB, K), lambda i, c0: (i, 0)),
    )
    return pl.pallas_call(
        _topk_win_body,
        grid_spec=grid_spec,
        out_shape=jax.ShapeDtypeStruct((N, K), jnp.int32),
    )(c0_blocks, xn, xn, xn, xn, xn, xn, batchf_col,
      batchf_row, batchf_row, batchf_row, batchf_row, batchf_row)


def _topk_body(xnb_ref, xn_ref, bfb_ref, bfr_ref, idx_ref):
    xnb = xnb_ref[...]
    xn = xn_ref[...]
    sqb = jnp.sum(xnb * xnb, axis=1, keepdims=True)
    ysq = xn * xn
    sqr = jnp.dot(jnp.ones((1, D), jnp.float32), ysq.T,
                  preferred_element_type=jnp.float32)
    dist = sqb + sqr - 2.0 * jnp.dot(xnb, xn.T, preferred_element_type=jnp.float32)
    cross = bfb_ref[...] != bfr_ref[...]
    d = jnp.where(cross, 1e10, dist)
    iota = lax.broadcasted_iota(jnp.int32, (RB, N), 1)
    inf = jnp.float32(jnp.inf)
    for k in range(K):
        m = jnp.min(d, axis=1, keepdims=True)
        amin = jnp.min(jnp.where(d == m, iota, N), axis=1, keepdims=True)
        idx_ref[:, k:k + 1] = amin
        d = jnp.where(iota == amin, inf, d)


def _topk(xn, batchf_col, batchf_row):
    return pl.pallas_call(
        _topk_body,
        grid=(N // RB,),
        in_specs=[
            pl.BlockSpec((RB, D), lambda i: (i, 0)),
            pl.BlockSpec((N, D), lambda i: (0, 0)),
            pl.BlockSpec((RB, 1), lambda i: (i, 0)),
            pl.BlockSpec((1, N), lambda i: (0, 0)),
        ],
        out_specs=pl.BlockSpec((RB, K), lambda i: (i, 0)),
        out_shape=jax.ShapeDtypeStruct((N, K), jnp.int32),
    )(xn, xn, batchf_col, batchf_row)


def _sc_gather_body(nrows, table_hbm, idx_hbm, out_hbm, idx_v, rows_v, sem):
    wid = lax.axis_index("s") * 2 + lax.axis_index("c")
    per_w = nrows // SC_WORKERS
    base = wid * per_w

    def chunk(c, _):
        off = pl.multiple_of(base + c * SC_CHUNK, SC_CHUNK)
        pltpu.sync_copy(idx_hbm.at[pl.ds(off, SC_CHUNK)], idx_v)
        pltpu.async_copy(table_hbm.at[idx_v], rows_v, sem).wait()
        pltpu.sync_copy(rows_v, out_hbm.at[pl.ds(off, SC_CHUNK)])
        return _

    lax.fori_loop(0, per_w // SC_CHUNK, chunk, None)


def _sc_gather(table, idx_flat):
    nrows = idx_flat.shape[0]
    mesh = plsc.VectorSubcoreMesh(core_axis_name="c", subcore_axis_name="s")
    kfn = functools.partial(
        pl.kernel,
        mesh=mesh,
        out_type=jax.ShapeDtypeStruct((nrows, D), jnp.float32),
        scratch_types=[
            pltpu.VMEM((SC_CHUNK,), jnp.int32),
            pltpu.VMEM((SC_CHUNK, D), jnp.float32),
            pltpu.SemaphoreType.DMA,
        ],
    )(functools.partial(_sc_gather_body, nrows))
    return kfn(table, idx_flat)


def _edge_body(a_ref, xj_ref, bfb_ref, w1b_ref, w2_ref, w3_ref,
               b1_ref, b2_ref, b3_ref, seg_ref, cnt_ref):
    @pl.when(pl.program_id(0) == 0)
    def _init():
        seg_ref[...] = jnp.zeros_like(seg_ref)
        cnt_ref[...] = jnp.zeros_like(cnt_ref)

    a = a_ref[...]
    a_rep = jnp.broadcast_to(a[:, None, :], (NB, K, BIG)).reshape(NB * K, BIG)
    bj = jnp.dot(xj_ref[...], w1b_ref[...], preferred_element_type=jnp.float32)
    h1 = jnp.maximum(a_rep + bj + b1_ref[...], 0.0)
    h2 = jnp.maximum(jnp.dot(h1, w2_ref[...], preferred_element_type=jnp.float32)
                     + b2_ref[...], 0.0)
    h3 = jnp.maximum(jnp.dot(h2, w3_ref[...], preferred_element_type=jnp.float32)
                     + b3_ref[...], 0.0)
    hx = jnp.mean(h3.reshape(NB, K, BIG), axis=1)
    gids = lax.broadcasted_iota(jnp.int32, (1, G), 1).astype(jnp.float32)
    onehot = (bfb_ref[...] == gids).astype(jnp.float32)
    seg_ref[...] += lax.dot_general(onehot, hx, (((0,), (0,)), ((), ())),
                                    preferred_element_type=jnp.float32)
    cnt_ref[...] += jnp.broadcast_to(
        jnp.sum(onehot, axis=0)[:, None], (G, BIG))


def _edge(A, XJ, batchf_col, W1b, W2, W3, b1, b2, b3):
    return pl.pallas_call(
        _edge_body,
        grid=(N // NB,),
        in_specs=[
            pl.BlockSpec((NB, BIG), lambda i: (i, 0)),
            pl.BlockSpec((NB * K, D), lambda i: (i, 0)),
            pl.BlockSpec((NB, 1), lambda i: (i, 0)),
            pl.BlockSpec((D, BIG), lambda i: (0, 0)),
            pl.BlockSpec((BIG, BIG), lambda i: (0, 0)),
            pl.BlockSpec((BIG, BIG), lambda i: (0, 0)),
            pl.BlockSpec((1, BIG), lambda i: (0, 0)),
            pl.BlockSpec((1, BIG), lambda i: (0, 0)),
            pl.BlockSpec((1, BIG), lambda i: (0, 0)),
        ],
        out_specs=(
            pl.BlockSpec((G, BIG), lambda i: (0, 0)),
            pl.BlockSpec((G, BIG), lambda i: (0, 0)),
        ),
        out_shape=(
            jax.ShapeDtypeStruct((G, BIG), jnp.float32),
            jax.ShapeDtypeStruct((G, BIG), jnp.float32),
        ),
    )(A, XJ, batchf_col, W1b, W2, W3,
      b1.reshape(1, BIG), b2.reshape(1, BIG), b3.reshape(1, BIG))


def _head_body(u_ref, gg_ref, gb_ref, seg_ref, cnt_ref,
               o1w_ref, o1b_ref, o2w_ref, o2b_ref, o3w_ref, o3b_ref, out_ref):
    u = u_ref[...]
    mu = jnp.mean(u, axis=0, keepdims=True)
    uc = u - mu
    var = jnp.mean(uc * uc, axis=0, keepdims=True)
    u1 = uc / jnp.sqrt(var + EPS) * gg_ref[...] + gb_ref[...]
    u2 = seg_ref[...] / jnp.maximum(cnt_ref[...], 1.0)
    uu = jnp.concatenate([u1, u2], axis=1)
    o = jnp.maximum(jnp.dot(uu, o1w_ref[...], preferred_element_type=jnp.float32)
                    + o1b_ref[...], 0.0)
    o = jnp.maximum(jnp.dot(o, o2w_ref[...], preferred_element_type=jnp.float32)
                    + o2b_ref[...], 0.0)
    out_ref[...] = jnp.dot(o, o3w_ref[...], preferred_element_type=jnp.float32) \
        + o3b_ref[...]


def _head(u, bng_gamma, bng_beta, seg, cnt, O1, o1, O2, o2, O3, o3):
    return pl.pallas_call(
        _head_body,
        out_shape=jax.ShapeDtypeStruct((G, OUT), jnp.float32),
    )(u, bng_gamma.reshape(1, GD), bng_beta.reshape(1, GD), seg, cnt,
      O1, o1.reshape(1, BIGGER), O2, o2.reshape(1, BIGGER), O3,
      o3.reshape(1, OUT))


def kernel(x, u, batch, bn_gamma, bn_beta, bng_gamma, bng_beta,
           W1, b1, W2, b2, W3, b3, O1, o1, O2, o2, O3, o3):
    batchf = batch.astype(jnp.float32)
    xn, A = _prep(x, W1, bn_gamma, bn_beta)

    gid = jnp.arange(G, dtype=jnp.int32)
    counts = jnp.sum(batch[None, :] == gid[:, None], axis=1)
    ends = jnp.cumsum(counts)
    starts = ends - counts
    rb_first = batch[::RB]
    rb_last = batch[RB - 1::RB]
    span = ends[rb_last] - starts[rb_first]
    win_ok = jnp.all(span <= WIN - WB) & jnp.all(counts >= K)
    c0_blocks = jnp.clip(starts[rb_first] // WB, 0, (N - WIN) // WB)

    bcol = batchf.reshape(N, 1)
    brow = batchf.reshape(1, N)
    c0b = c0_blocks.astype(jnp.int32)
    idx = lax.cond(
        win_ok,
        lambda: _topk_win(xn, bcol, brow, c0b),
        lambda: _topk(xn, bcol, brow),
    )
    XJ = _sc_gather(xn, idx.reshape(N * K))
    seg, cnt = _edge(A, XJ, bcol, W1[D:, :], W2, W3, b1, b2, b3)
    return _head(u, bng_gamma, bng_beta, seg, cnt, O1, o1, O2, o2, O3, o3)
